# baseline (speedup 1.0000x reference)
import jax
import jax.numpy as jnp
from jax.experimental import pallas as pl
from jax.experimental.pallas import tpu as pltpu
from jax.experimental.pallas import tpu_sc as plsc

N = 10000
D = 128
GAMMA = 3.0
T = 1000
K = 5

BM = 400

P_PAIRS = 5 * T + 5 * T + 5 * T + T + T
GWIN = 128
P_PAD = 17024


CHUNKS = [(0, 2048), (2048, 2048), (4096, 2048), (6144, 2048), (8192, 1808)]


def _mm1_body(s_ref, w_ref, h_ref):
    h = jnp.zeros((BM, D), jnp.float32)
    for c0, cw in CHUNKS:
        s_c = s_ref[:, c0:c0 + cw].astype(jnp.bfloat16)
        w_c = w_ref[c0:c0 + cw, :].astype(jnp.bfloat16)
        h = h + jnp.dot(s_c, w_c, preferred_element_type=jnp.float32)
    h_ref[...] = jnp.maximum(h, 0.0).astype(jnp.bfloat16)


def _mm2_body(s_ref, h_ref, o_ref):
    o = jnp.zeros((BM, D), jnp.float32)
    for c0, cw in CHUNKS:
        s_c = s_ref[:, c0:c0 + cw].astype(jnp.bfloat16)
        h_c = h_ref[c0:c0 + cw, :]
        o = o + jnp.dot(s_c, h_c, preferred_element_type=jnp.float32)
    o_ref[...] = o


def _gcn_out(support, W0):
    cp = pltpu.CompilerParams(dimension_semantics=("parallel",))
    hidden = pl.pallas_call(
        _mm1_body,
        grid=(N // BM,),
        in_specs=[
            pl.BlockSpec((BM, N), lambda mi: (mi, 0)),
            pl.BlockSpec((N, D), lambda mi: (0, 0)),
        ],
        out_specs=pl.BlockSpec((BM, D), lambda mi: (mi, 0)),
        out_shape=jax.ShapeDtypeStruct((N, D), jnp.bfloat16),
        compiler_params=cp,
    )(support, W0)
    return pl.pallas_call(
        _mm2_body,
        grid=(N // BM,),
        in_specs=[
            pl.BlockSpec((BM, N), lambda mi: (mi, 0)),
            pl.BlockSpec((N, D), lambda mi: (0, 0)),
        ],
        out_specs=pl.BlockSpec((BM, D), lambda mi: (mi, 0)),
        out_shape=jax.ShapeDtypeStruct((N, D), jnp.float32),
        compiler_params=cp,
    )(support, hidden)


def _sc_gather(out_hbm, left, right):
    pad = jnp.zeros((P_PAD - P_PAIRS,), jnp.int32)
    left2 = jnp.concatenate([left, pad]).reshape(1, P_PAD)
    right2 = jnp.concatenate([right, pad]).reshape(1, P_PAD)
    mesh = plsc.VectorSubcoreMesh(core_axis_name="core",
                                  subcore_axis_name="subcore")
    row_t = jax.ShapeDtypeStruct((P_PAD, D), jnp.float32)

    @pl.kernel(out_type=[row_t, row_t], mesh=mesh)
    def k(x_hbm, li_hbm, ri_hbm, lo_hbm, ro_hbm):
        def body(li_vmem, ri_vmem, lo_vmem, ro_vmem):
            pltpu.sync_copy(x_hbm.at[li_vmem.at[0]], lo_vmem)
            pltpu.sync_copy(x_hbm.at[ri_vmem.at[0]], ro_vmem)

        pltpu.emit_pipeline(
            body,
            grid=(P_PAD // GWIN,),
            in_specs=[pl.BlockSpec((1, GWIN), lambda i: (0, i)),
                      pl.BlockSpec((1, GWIN), lambda i: (0, i))],
            out_specs=[pl.BlockSpec((GWIN, D), lambda i: (i, 0)),
                       pl.BlockSpec((GWIN, D), lambda i: (i, 0))],
            core_axis_name=("core", "subcore"),
            dimension_semantics=(pltpu.PARALLEL,),
        )(li_hbm, ri_hbm, lo_hbm, ro_hbm)

    return k(out_hbm, left2, right2)


def _combine_body(l_ref, r_ref, o_ref):
    d = jnp.sum(jnp.abs(l_ref[...] - r_ref[...]), axis=1, keepdims=True)
    dA5 = d[0:5000]
    dB1 = d[5000:10000]
    dB2 = d[10000:15000]
    dA2 = d[15000:16000]
    dB3 = d[16000:17000]
    t1 = jnp.sum(jnp.maximum(dA5 + GAMMA - dB1, 0.0))
    t2 = jnp.sum(jnp.maximum(dA5 + GAMMA - dB2, 0.0))
    t3 = jnp.sum(jnp.maximum(dA2 + GAMMA - dB3, 0.0))
    o_ref[0, 0] = (t1 + t2 + t3) / (2 * K * T + T)


def _combine(L, R):
    return pl.pallas_call(
        _combine_body,
        out_shape=jax.ShapeDtypeStruct((1, 1), jnp.float32),
        out_specs=pl.BlockSpec(memory_space=pltpu.SMEM),
    )(L, R)


def kernel(features, support, W0, ILL0, ILL1, neg_left, neg_right,
           neg2_left, neg2_right, feedback_neg_left, feedback_neg_right,
           feedback_pos_left, feedback_pos_right):
    out = _gcn_out(support, W0)
    left = jnp.concatenate([
        jnp.repeat(ILL0, K), neg_left, neg2_left,
        feedback_pos_left, feedback_neg_left]).astype(jnp.int32)
    right = jnp.concatenate([
        jnp.repeat(ILL1, K), neg_right, neg2_right,
        feedback_pos_right, feedback_neg_right]).astype(jnp.int32)
    L, R = _sc_gather(out, left, right)
    return _combine(L, R)[0, 0]


# speedup vs baseline: 1.0296x; 1.0296x over previous
import jax
import jax.numpy as jnp
from jax.experimental import pallas as pl
from jax.experimental.pallas import tpu as pltpu
from jax.experimental.pallas import tpu_sc as plsc

N = 10000
D = 128
GAMMA = 3.0
T = 1000
K = 5

BM = 400

P_PAIRS = T + K * T + K * T + T + T
GWIN = 128
P_PAD = 13056

CHUNKS = [(0, 2048), (2048, 2048), (4096, 2048), (6144, 2048), (8192, 1808)]


def _mm_body(s_ref, w_ref, o_ref, hidden_ref):
    p = pl.program_id(0)
    mi = pl.program_id(1)

    @pl.when(p == 0)
    def _():
        h = jnp.zeros((BM, D), jnp.float32)
        for c0, cw in CHUNKS:
            s_c = s_ref[:, c0:c0 + cw].astype(jnp.bfloat16)
            w_c = w_ref[c0:c0 + cw, :].astype(jnp.bfloat16)
            h = h + jnp.dot(s_c, w_c, preferred_element_type=jnp.float32)
        hidden_ref[pl.ds(mi * BM, BM), :] = jnp.maximum(
            h, 0.0).astype(jnp.bfloat16)

    @pl.when(p == 1)
    def _():
        o = jnp.zeros((BM, D), jnp.float32)
        for c0, cw in CHUNKS:
            s_c = s_ref[:, c0:c0 + cw].astype(jnp.bfloat16)
            h_c = hidden_ref[c0:c0 + cw, :]
            o = o + jnp.dot(s_c, h_c, preferred_element_type=jnp.float32)
        o_ref[...] = o


def _gcn_out(support, W0):
    return pl.pallas_call(
        _mm_body,
        grid=(2, N // BM),
        in_specs=[
            pl.BlockSpec((BM, N), lambda p, mi: (mi, 0)),
            pl.BlockSpec((N, D), lambda p, mi: (0, 0)),
        ],
        out_specs=pl.BlockSpec((BM, D), lambda p, mi: (mi, 0)),
        out_shape=jax.ShapeDtypeStruct((N, D), jnp.float32),
        scratch_shapes=[
            pltpu.VMEM((N, D), jnp.bfloat16),
        ],
    )(support, W0)


def _sc_gather(out_hbm, left, right):
    pad = jnp.zeros((P_PAD - P_PAIRS,), jnp.int32)
    left2 = jnp.concatenate([left, pad]).reshape(1, P_PAD)
    right2 = jnp.concatenate([right, pad]).reshape(1, P_PAD)
    mesh = plsc.VectorSubcoreMesh(core_axis_name="core",
                                  subcore_axis_name="subcore")
    row_t = jax.ShapeDtypeStruct((P_PAD, D), jnp.float32)

    @pl.kernel(out_type=[row_t, row_t], mesh=mesh)
    def k(x_hbm, li_hbm, ri_hbm, lo_hbm, ro_hbm):
        def body(li_vmem, ri_vmem, lo_vmem, ro_vmem):
            pltpu.sync_copy(x_hbm.at[li_vmem.at[0]], lo_vmem)
            pltpu.sync_copy(x_hbm.at[ri_vmem.at[0]], ro_vmem)

        pltpu.emit_pipeline(
            body,
            grid=(P_PAD // GWIN,),
            in_specs=[pl.BlockSpec((1, GWIN), lambda i: (0, i)),
                      pl.BlockSpec((1, GWIN), lambda i: (0, i))],
            out_specs=[pl.BlockSpec((GWIN, D), lambda i: (i, 0)),
                       pl.BlockSpec((GWIN, D), lambda i: (i, 0))],
            core_axis_name=("core", "subcore"),
            dimension_semantics=(pltpu.PARALLEL,),
        )(li_hbm, ri_hbm, lo_hbm, ro_hbm)

    return k(out_hbm, left2, right2)


def _combine_body(l_ref, r_ref, o_ref):
    d = jnp.sum(jnp.abs(l_ref[...] - r_ref[...]), axis=1, keepdims=True)
    dA = d[0:T]
    dA2 = d[11 * T:12 * T]
    dB3 = d[12 * T:13 * T]
    acc = jnp.sum(jnp.maximum(dA2 + GAMMA - dB3, 0.0))
    for k in range(2 * K):
        dBk = d[(1 + k) * T:(2 + k) * T]
        acc = acc + jnp.sum(jnp.maximum(dA + GAMMA - dBk, 0.0))
    o_ref[0, 0] = acc / (2 * K * T + T)


def _combine(L, R):
    return pl.pallas_call(
        _combine_body,
        out_shape=jax.ShapeDtypeStruct((1, 1), jnp.float32),
        out_specs=pl.BlockSpec(memory_space=pltpu.SMEM),
    )(L, R)


def kernel(features, support, W0, ILL0, ILL1, neg_left, neg_right,
           neg2_left, neg2_right, feedback_neg_left, feedback_neg_right,
           feedback_pos_left, feedback_pos_right):
    out = _gcn_out(support, W0)
    nl = neg_left.reshape(T, K).T.reshape(K * T)
    nr = neg_right.reshape(T, K).T.reshape(K * T)
    n2l = neg2_left.reshape(T, K).T.reshape(K * T)
    n2r = neg2_right.reshape(T, K).T.reshape(K * T)
    left = jnp.concatenate([
        ILL0, nl, n2l, feedback_pos_left, feedback_neg_left]).astype(jnp.int32)
    right = jnp.concatenate([
        ILL1, nr, n2r, feedback_pos_right, feedback_neg_right]).astype(jnp.int32)
    L, R = _sc_gather(out, left, right)
    return _combine(L, R)[0, 0]
